# Initial kernel scaffold; baseline (speedup 1.0000x reference)
import functools

import jax
import jax.numpy as jnp
from jax import lax
from jax.experimental import pallas as pl
from jax.experimental.pallas import tpu as pltpu
from jax.experimental.pallas import tpu_sc as plsc

E = 16384
L = 50
D = 64
NC = 2
NS = 16
NW = NC * NS
EPW = E // NW
LANE = 16
NG = EPW // LANE


def _sc_body(tok_hbm, emb_hbm, out_hbm, tok_v, emb_v, c_v, out_v):
    wid = lax.axis_index("s") * NC + lax.axis_index("c")
    base_e = wid * EPW
    pltpu.sync_copy(tok_hbm.at[pl.ds(base_e, EPW)], tok_v)
    pltpu.sync_copy(emb_hbm, emb_v)

    lane = lax.iota(jnp.int32, LANE)
    inv_l = 1.0 / L
    pb = [emb_v[0, pl.ds(16 * j, 16)] for j in range(4)]
    d1 = [(emb_v[1, pl.ds(16 * j, 16)] - pb[j]) * inv_l for j in range(4)]
    d2 = [(emb_v[2, pl.ds(16 * j, 16)] - pb[j]) * inv_l for j in range(4)]
    d3 = [(emb_v[3, pl.ds(16 * j, 16)] - pb[j]) * inv_l for j in range(4)]

    def group_body(g, carry):
        rows = g * LANE + lane
        sb0 = jnp.zeros((LANE,), jnp.int32)
        sb1 = jnp.zeros((LANE,), jnp.int32)
        c3 = jnp.zeros((LANE,), jnp.int32)
        for l in range(L):
            col = jnp.full((LANE,), l, jnp.int32)
            t = plsc.load_gather(tok_v, [rows, col])
            b0 = t & 1
            b1 = t >> 1
            sb0 = sb0 + b0
            sb1 = sb1 + b1
            c3 = c3 + (b0 & b1)
        c1 = sb0 - c3
        c2 = sb1 - c3
        c_v[pl.ds(0, 16)] = c1.astype(jnp.float32)
        c_v[pl.ds(16, 16)] = c2.astype(jnp.float32)
        c_v[pl.ds(32, 16)] = c3.astype(jnp.float32)
        for e in range(LANE):
            idx = jnp.full((LANE,), e, jnp.int32)
            bc1 = plsc.load_gather(c_v, [idx])
            bc2 = plsc.load_gather(c_v, [idx + 16])
            bc3 = plsc.load_gather(c_v, [idx + 32])
            ge = g * LANE + e
            for j in range(4):
                out_v[ge, pl.ds(16 * j, 16)] = (
                    pb[j] + bc1 * d1[j] + bc2 * d2[j] + bc3 * d3[j])
        return carry

    lax.fori_loop(0, NG, group_body, 0)
    pltpu.sync_copy(out_v, out_hbm.at[pl.ds(base_e, EPW)])


_sc_call = functools.partial(
    pl.kernel,
    out_type=jax.ShapeDtypeStruct((E, D), jnp.float32),
    mesh=plsc.VectorSubcoreMesh(
        core_axis_name="c", subcore_axis_name="s",
        num_cores=NC, num_subcores=NS),
    scratch_types=[
        pltpu.VMEM((EPW, L), jnp.int32),
        pltpu.VMEM((4, D), jnp.float32),
        pltpu.VMEM((48,), jnp.float32),
        pltpu.VMEM((EPW, D), jnp.float32),
    ],
)(_sc_body)


def kernel(overlap_similarity, overlap_length, reads_tokens, emb_table, W, b):
    return _sc_call(reads_tokens, emb_table)


# speedup vs baseline: 74.9524x; 74.9524x over previous
import functools

import jax
import jax.numpy as jnp
from jax import lax
from jax.experimental import pallas as pl
from jax.experimental.pallas import tpu as pltpu
from jax.experimental.pallas import tpu_sc as plsc

E = 16384
L = 50
D = 64
NC = 2
NS = 16
NW = NC * NS
EPW = E // NW
LANE = 16
NG = EPW // LANE


def _sc_body(tok_hbm, emb_hbm, out_hbm, tok_v, emb_v, c_v, out_v):
    wid = lax.axis_index("s") * NC + lax.axis_index("c")
    base_e = wid * EPW
    pltpu.sync_copy(tok_hbm.at[pl.ds(base_e * L, EPW * L)], tok_v)
    pltpu.sync_copy(emb_hbm, emb_v)

    lane = lax.iota(jnp.int32, LANE)
    lane_l = lane * L
    inv_l = 1.0 / L
    pb = [emb_v[0, pl.ds(16 * j, 16)] for j in range(4)]
    d1 = [(emb_v[1, pl.ds(16 * j, 16)] - pb[j]) * inv_l for j in range(4)]
    d2 = [(emb_v[2, pl.ds(16 * j, 16)] - pb[j]) * inv_l for j in range(4)]
    d3 = [(emb_v[3, pl.ds(16 * j, 16)] - pb[j]) * inv_l for j in range(4)]

    def group_body(g, carry):
        idx = g * (LANE * L) + lane_l
        sb0 = jnp.zeros((LANE,), jnp.int32)
        sb1 = jnp.zeros((LANE,), jnp.int32)
        c3 = jnp.zeros((LANE,), jnp.int32)
        for l in range(L):
            t = plsc.load_gather(tok_v, [idx + l])
            b0 = t & 1
            b1 = t >> 1
            sb0 = sb0 + b0
            sb1 = sb1 + b1
            c3 = c3 + (b0 & b1)
        c1 = sb0 - c3
        c2 = sb1 - c3
        c_v[pl.ds(16, 16)] = c1.astype(jnp.float32)
        c_v[pl.ds(32, 16)] = c2.astype(jnp.float32)
        c_v[pl.ds(48, 16)] = c3.astype(jnp.float32)
        for e in range(LANE):
            idx = jnp.full((LANE,), 16 + e, jnp.int32)
            bc1 = plsc.load_gather(c_v, [idx])
            bc2 = plsc.load_gather(c_v, [idx + 16])
            bc3 = plsc.load_gather(c_v, [idx + 32])
            ge = g * LANE + e
            for j in range(4):
                out_v[ge, pl.ds(16 * j, 16)] = (
                    pb[j] + bc1 * d1[j] + bc2 * d2[j] + bc3 * d3[j])
        return carry

    lax.fori_loop(0, NG, group_body, 0)
    pltpu.sync_copy(out_v, out_hbm.at[pl.ds(base_e, EPW)])


_sc_call = functools.partial(
    pl.kernel,
    out_type=jax.ShapeDtypeStruct((E, D), jnp.float32),
    mesh=plsc.VectorSubcoreMesh(
        core_axis_name="c", subcore_axis_name="s",
        num_cores=NC, num_subcores=NS),
    compiler_params=pltpu.CompilerParams(needs_layout_passes=False),
    scratch_types=[
        pltpu.VMEM((EPW * L,), jnp.int32),
        pltpu.VMEM((4, D), jnp.float32),
        pltpu.VMEM((64,), jnp.float32),
        pltpu.VMEM((EPW, D), jnp.float32),
    ],
)(_sc_body)


def kernel(overlap_similarity, overlap_length, reads_tokens, emb_table, W, b):
    return _sc_call(reads_tokens.reshape(E * L), emb_table)


# speedup vs baseline: 77.6924x; 1.0366x over previous
import functools

import jax
import jax.numpy as jnp
from jax import lax
from jax.experimental import pallas as pl
from jax.experimental.pallas import tpu as pltpu
from jax.experimental.pallas import tpu_sc as plsc

E = 16384
L = 50
D = 64
NC = 2
NS = 16
NW = NC * NS
EPW = E // NW
LANE = 16
NG = EPW // LANE


def _sc_body(tok_hbm, c1_hbm, c2_hbm, c3_hbm, tok_v, c1_v, c2_v, c3_v):
    wid = lax.axis_index("s") * NC + lax.axis_index("c")
    base_e = wid * EPW
    pltpu.sync_copy(tok_hbm.at[pl.ds(base_e, EPW)], tok_v)

    lane = lax.iota(jnp.int32, LANE)

    def group_body(g, carry):
        rows = g * LANE + lane
        sb0 = jnp.zeros((LANE,), jnp.int32)
        sb1 = jnp.zeros((LANE,), jnp.int32)
        c3 = jnp.zeros((LANE,), jnp.int32)
        for l in range(L):
            col = jnp.full((LANE,), l, jnp.int32)
            t = plsc.load_gather(tok_v, [rows, col])
            b0 = t & 1
            b1 = t >> 1
            sb0 = sb0 + b0
            sb1 = sb1 + b1
            c3 = c3 + (b0 & b1)
        gs = g * LANE
        c1_v[pl.ds(gs, 16)] = (sb0 - c3).astype(jnp.float32)
        c2_v[pl.ds(gs, 16)] = (sb1 - c3).astype(jnp.float32)
        c3_v[pl.ds(gs, 16)] = c3.astype(jnp.float32)
        return carry

    lax.fori_loop(0, NG, group_body, 0)
    pltpu.sync_copy(c1_v, c1_hbm.at[pl.ds(base_e, EPW)])
    pltpu.sync_copy(c2_v, c2_hbm.at[pl.ds(base_e, EPW)])
    pltpu.sync_copy(c3_v, c3_hbm.at[pl.ds(base_e, EPW)])


_sc_counts = functools.partial(
    pl.kernel,
    out_type=[jax.ShapeDtypeStruct((E,), jnp.float32)] * 3,
    mesh=plsc.VectorSubcoreMesh(
        core_axis_name="c", subcore_axis_name="s",
        num_cores=NC, num_subcores=NS),
    compiler_params=pltpu.CompilerParams(needs_layout_passes=False),
    scratch_types=[
        pltpu.VMEM((EPW, L), jnp.int32),
        pltpu.VMEM((EPW,), jnp.float32),
        pltpu.VMEM((EPW,), jnp.float32),
        pltpu.VMEM((EPW,), jnp.float32),
    ],
)(_sc_body)


BE = 2048


def _tc_body(c1_ref, c2_ref, c3_ref, emb_ref, out_ref):
    emb = emb_ref[...]
    pb = emb[0:1, :]
    inv_l = 1.0 / L
    d1 = (emb[1:2, :] - pb) * inv_l
    d2 = (emb[2:3, :] - pb) * inv_l
    d3 = (emb[3:4, :] - pb) * inv_l
    c1 = c1_ref[...][:, None]
    c2 = c2_ref[...][:, None]
    c3 = c3_ref[...][:, None]
    out_ref[...] = pb + c1 * d1 + c2 * d2 + c3 * d3


def _tc_expand(c1, c2, c3, emb_table):
    return pl.pallas_call(
        _tc_body,
        grid=(E // BE,),
        in_specs=[
            pl.BlockSpec((BE,), lambda i: (i,)),
            pl.BlockSpec((BE,), lambda i: (i,)),
            pl.BlockSpec((BE,), lambda i: (i,)),
            pl.BlockSpec((4, D), lambda i: (0, 0)),
        ],
        out_specs=pl.BlockSpec((BE, D), lambda i: (i, 0)),
        out_shape=jax.ShapeDtypeStruct((E, D), jnp.float32),
    )(c1, c2, c3, emb_table)


def kernel(overlap_similarity, overlap_length, reads_tokens, emb_table, W, b):
    c1, c2, c3 = _sc_counts(reads_tokens)
    return _tc_expand(c1, c2, c3, emb_table)


# speedup vs baseline: 140.1226x; 1.8036x over previous
import functools

import jax
import jax.numpy as jnp
from jax import lax
from jax.experimental import pallas as pl
from jax.experimental.pallas import tpu as pltpu
from jax.experimental.pallas import tpu_sc as plsc

E = 16384
L = 50
D = 64
NC = 2
NS = 16
NW = NC * NS
EPW = E // NW
LANE = 16
NG = EPW // LANE


def _sc_body(tok_hbm, c1_hbm, c2_hbm, c3_hbm, tok_v, c1_v, c2_v, c3_v):
    wid = lax.axis_index("s") * NC + lax.axis_index("c")
    base_e = wid * EPW
    pltpu.sync_copy(tok_hbm.at[:, pl.ds(base_e, EPW)], tok_v)

    def group_body(g, carry):
        gs = g * LANE
        sb0 = jnp.zeros((LANE,), jnp.int32)
        sb1 = jnp.zeros((LANE,), jnp.int32)
        c3 = jnp.zeros((LANE,), jnp.int32)
        for l in range(L):
            t = tok_v[l, pl.ds(gs, LANE)]
            b0 = t & 1
            b1 = t >> 1
            sb0 = sb0 + b0
            sb1 = sb1 + b1
            c3 = c3 + (b0 & b1)
        c1_v[pl.ds(gs, LANE)] = (sb0 - c3).astype(jnp.float32)
        c2_v[pl.ds(gs, LANE)] = (sb1 - c3).astype(jnp.float32)
        c3_v[pl.ds(gs, LANE)] = c3.astype(jnp.float32)
        return carry

    lax.fori_loop(0, NG, group_body, 0)
    pltpu.sync_copy(c1_v, c1_hbm.at[pl.ds(base_e, EPW)])
    pltpu.sync_copy(c2_v, c2_hbm.at[pl.ds(base_e, EPW)])
    pltpu.sync_copy(c3_v, c3_hbm.at[pl.ds(base_e, EPW)])


_sc_counts = functools.partial(
    pl.kernel,
    out_type=[jax.ShapeDtypeStruct((E,), jnp.float32)] * 3,
    mesh=plsc.VectorSubcoreMesh(
        core_axis_name="c", subcore_axis_name="s",
        num_cores=NC, num_subcores=NS),
    compiler_params=pltpu.CompilerParams(needs_layout_passes=False),
    scratch_types=[
        pltpu.VMEM((L, EPW), jnp.int32),
        pltpu.VMEM((EPW,), jnp.float32),
        pltpu.VMEM((EPW,), jnp.float32),
        pltpu.VMEM((EPW,), jnp.float32),
    ],
)(_sc_body)


BE = 2048


def _tc_body(c1_ref, c2_ref, c3_ref, embt_ref, outt_ref):
    embt = embt_ref[...]
    pb = embt[:, 0:1]
    inv_l = 1.0 / L
    d1 = (embt[:, 1:2] - pb) * inv_l
    d2 = (embt[:, 2:3] - pb) * inv_l
    d3 = (embt[:, 3:4] - pb) * inv_l
    c1 = c1_ref[...][None, :]
    c2 = c2_ref[...][None, :]
    c3 = c3_ref[...][None, :]
    outt_ref[...] = pb + d1 * c1 + d2 * c2 + d3 * c3


def _tc_expand(c1, c2, c3, embt):
    return pl.pallas_call(
        _tc_body,
        grid=(E // BE,),
        in_specs=[
            pl.BlockSpec((BE,), lambda i: (i,)),
            pl.BlockSpec((BE,), lambda i: (i,)),
            pl.BlockSpec((BE,), lambda i: (i,)),
            pl.BlockSpec((D, 4), lambda i: (0, 0)),
        ],
        out_specs=pl.BlockSpec((D, BE), lambda i: (0, i)),
        out_shape=jax.ShapeDtypeStruct((D, E), jnp.float32),
    )(c1, c2, c3, embt)


def kernel(overlap_similarity, overlap_length, reads_tokens, emb_table, W, b):
    c1, c2, c3 = _sc_counts(reads_tokens.T)
    return _tc_expand(c1, c2, c3, emb_table.T).T
